# Initial kernel scaffold; baseline (speedup 1.0000x reference)
import jax
import jax.numpy as jnp
from jax.experimental import pallas as pl


def kernel(x, h, router_weight, W_q, K_all, V_all):
    raise NotImplementedError("write your pallas kernel here")


# speedup vs baseline: 51.2567x; 51.2567x over previous
import functools
import math

import jax
import jax.numpy as jnp
from jax import lax
from jax.experimental import pallas as pl
from jax.experimental.pallas import tpu as pltpu
from jax.experimental.pallas import tpu_sc as plsc

B, S, D, R, NK = 4, 128, 256, 64, 100000
CK, FK = 20, 10
T = B * S
TILE = 128
NT = 784
NKP = NT * TILE
TNB = 2048
GB = TNB // TILE
NSTEPS = NKP // TNB
SEL_T = 24
POOL = SEL_T * TILE
TB = 128
NEG = float("-inf")
IMAX = 2**31 - 1


NTP = 896


def _router_body(x_ref, w_ref, logits_ref, sel_ref, mx_ref, buf_ref):
    step = pl.program_id(0)
    logits = jnp.dot(x_ref[...], w_ref[...], preferred_element_type=jnp.float32)
    logits_ref[...] = logits
    col = step * TNB + lax.broadcasted_iota(jnp.int32, (1, TNB), 1)
    lv = jnp.where(col < NK, logits, NEG)
    gmax = jnp.concatenate(
        [jnp.max(lv[:, g * TILE:(g + 1) * TILE], axis=-1, keepdims=True)
         for g in range(GB)], axis=1)
    lane = lax.broadcasted_iota(jnp.int32, (T, TILE), 1)
    tile8 = jnp.concatenate([gmax] * (TILE // GB), axis=1)
    buf_ref[...] = jnp.where(lane // GB == step % (TILE // GB),
                             tile8, buf_ref[...])
    mx_ref[:, pl.ds(pl.multiple_of((step // (TILE // GB)) * TILE, TILE),
                    TILE)] = buf_ref[...]

    @pl.when(step == NSTEPS - 1)
    def _select():
        tid = lax.broadcasted_iota(jnp.int32, (T, NTP), 1)
        its = lax.broadcasted_iota(jnp.int32, (T, SEL_T), 1)
        mx0 = jnp.where(tid < NT, mx_ref[...], NEG)

        def body(j, carry):
            mx, acc = carry
            m = jnp.max(mx, axis=-1, keepdims=True)
            s = jnp.min(jnp.where(mx == m, tid, IMAX), axis=-1, keepdims=True)
            acc = jnp.where(its == j, s, acc)
            mx = jnp.where(tid == s, NEG, mx)
            return mx, acc

        _, selcols = lax.fori_loop(
            0, SEL_T, body, (mx0, jnp.zeros((T, SEL_T), jnp.int32)))
        rowbase = lax.broadcasted_iota(jnp.int32, (T, SEL_T), 0) * NT
        sel_ref[...] = rowbase + selcols


def _run_router(x2, rw):
    return pl.pallas_call(
        _router_body,
        grid=(NSTEPS,),
        in_specs=[
            pl.BlockSpec((T, D), lambda i: (0, 0)),
            pl.BlockSpec((D, TNB), lambda i: (0, i)),
        ],
        out_specs=[
            pl.BlockSpec((T, TNB), lambda i: (0, i)),
            pl.BlockSpec((T, SEL_T), lambda i: (0, 0)),
        ],
        out_shape=[
            jax.ShapeDtypeStruct((T, NKP), jnp.float32),
            jax.ShapeDtypeStruct((T, SEL_T), jnp.int32),
        ],
        scratch_shapes=[pltpu.VMEM((T, NTP), jnp.float32),
                        pltpu.VMEM((T, TILE), jnp.float32)],
        compiler_params=pltpu.CompilerParams(
            dimension_semantics=("arbitrary",)),
    )(x2, rw)


_info = plsc.get_sparse_core_info()
_NC, _NS = _info.num_cores, _info.num_subcores
_NW = _NC * _NS


def _sc_gather(table, idx, rows_total, d):
    bpw = rows_total // _NW
    mesh = plsc.VectorSubcoreMesh(core_axis_name="c", subcore_axis_name="s")

    @functools.partial(
        pl.kernel, mesh=mesh,
        out_type=jax.ShapeDtypeStruct((rows_total, d), jnp.float32),
        scratch_types=[
            pltpu.VMEM((bpw,), jnp.int32),
            pltpu.VMEM((bpw, d), jnp.float32),
            pltpu.SemaphoreType.DMA,
        ],
    )
    def gk(table_hbm, idx_hbm, out_hbm, idx_v, rows_v, sem):
        wid = lax.axis_index("s") * _NC + lax.axis_index("c")
        base = wid * bpw
        pltpu.sync_copy(idx_hbm.at[pl.ds(base, bpw)], idx_v)
        off = 0
        while off < bpw:
            cs = min(128, bpw - off)
            pltpu.async_copy(table_hbm.at[idx_v.at[pl.ds(off, cs)]],
                             rows_v.at[pl.ds(off, cs)], sem).wait()
            off += cs
        pltpu.sync_copy(rows_v, out_hbm.at[pl.ds(base, bpw)])

    return gk(table, idx)


def _coarse_body(pool_ref, sel_ref, h_ref, wqt_ref, cs_ref, ci_ref, q_ref):
    t0 = pl.program_id(0) * TB
    tglob = t0 + lax.broadcasted_iota(jnp.int32, (TB, 1), 0)
    tau = sel_ref[...] - tglob * NT
    off128 = lax.broadcasted_iota(jnp.int32, (TB, TILE), 1)
    gid = jnp.concatenate(
        [tau[:, j:j + 1] * TILE + off128 for j in range(SEL_T)], axis=1)
    vals = jnp.where(gid < NK, pool_ref[...], NEG)
    itc = lax.broadcasted_iota(jnp.int32, (TB, CK), 1)

    def body(j, carry):
        vals, cs, ci = carry
        m = jnp.max(vals, axis=-1, keepdims=True)
        s = jnp.min(jnp.where(vals == m, gid, IMAX), axis=-1, keepdims=True)
        cs = jnp.where(itc == j, m, cs)
        ci = jnp.where(itc == j, s, ci)
        vals = jnp.where(gid == s, NEG, vals)
        return vals, cs, ci

    _, cs, ci = lax.fori_loop(
        0, CK, body,
        (vals, jnp.zeros((TB, CK), jnp.float32), jnp.zeros((TB, CK), jnp.int32)))
    cs_ref[...] = cs
    ci_ref[...] = ci
    q_ref[...] = jnp.dot(h_ref[...], wqt_ref[...],
                         preferred_element_type=jnp.float32)


def _run_coarse(pool, selflat, h2, wqt):
    return pl.pallas_call(
        _coarse_body,
        grid=(T // TB,),
        in_specs=[
            pl.BlockSpec((TB, POOL), lambda i: (i, 0)),
            pl.BlockSpec((TB, SEL_T), lambda i: (i, 0)),
            pl.BlockSpec((TB, R), lambda i: (i, 0)),
            pl.BlockSpec((R, R), lambda i: (0, 0)),
        ],
        out_specs=[
            pl.BlockSpec((TB, CK), lambda i: (i, 0)),
            pl.BlockSpec((TB, CK), lambda i: (i, 0)),
            pl.BlockSpec((TB, R), lambda i: (i, 0)),
        ],
        out_shape=[
            jax.ShapeDtypeStruct((T, CK), jnp.float32),
            jax.ShapeDtypeStruct((T, CK), jnp.int32),
            jax.ShapeDtypeStruct((T, R), jnp.float32),
        ],
    )(pool, selflat, h2, wqt)


def _fine_body(q_ref, kc_ref, ci_ref, fw_ref, fi_ref):
    q = q_ref[...]
    kc = kc_ref[...]
    ci = ci_ref[...]
    parts = []
    for j in range(CK):
        odd = (ci[:, j:j + 1] % 2) == 1
        kj = jnp.where(odd, kc[:, j * 2 * R + R:(j + 1) * 2 * R],
                       kc[:, j * 2 * R:j * 2 * R + R])
        parts.append(jnp.sum(kj * q, axis=-1, keepdims=True))
    sc = jnp.concatenate(parts, axis=1) / math.sqrt(R)
    it20 = lax.broadcasted_iota(jnp.int32, (TB, CK), 1)
    it10 = lax.broadcasted_iota(jnp.int32, (TB, FK), 1)
    def body(j, carry):
        sc, fs, fi = carry
        m = jnp.max(sc, axis=-1, keepdims=True)
        p = jnp.min(jnp.where(sc == m, it20, IMAX), axis=-1, keepdims=True)
        g = jnp.min(jnp.where(it20 == p, ci, IMAX), axis=-1, keepdims=True)
        fs = jnp.where(it10 == j, m, fs)
        fi = jnp.where(it10 == j, g, fi)
        sc = jnp.where(it20 == p, NEG, sc)
        return sc, fs, fi

    _, fs, fi = lax.fori_loop(
        0, FK, body,
        (sc, jnp.zeros((TB, FK), jnp.float32), jnp.zeros((TB, FK), jnp.int32)))
    e = jnp.exp(fs - fs[:, 0:1])
    fw_ref[...] = e / jnp.sum(e, axis=-1, keepdims=True)
    fi_ref[...] = fi


def _run_fine(q, kc, ci):
    return pl.pallas_call(
        _fine_body,
        grid=(T // TB,),
        in_specs=[
            pl.BlockSpec((TB, R), lambda i: (i, 0)),
            pl.BlockSpec((TB, CK * 2 * R), lambda i: (i, 0)),
            pl.BlockSpec((TB, CK), lambda i: (i, 0)),
        ],
        out_specs=[
            pl.BlockSpec((TB, FK), lambda i: (i, 0)),
            pl.BlockSpec((TB, FK), lambda i: (i, 0)),
        ],
        out_shape=[
            jax.ShapeDtypeStruct((T, FK), jnp.float32),
            jax.ShapeDtypeStruct((T, FK), jnp.int32),
        ],
    )(q, kc, ci)


def _combine_body(rows_ref, fw_ref, out_ref):
    rows = rows_ref[...]
    fw = fw_ref[...]
    acc = jnp.zeros((TB, D), jnp.float32)
    for j in range(FK):
        acc = acc + rows[:, j * D:(j + 1) * D] * fw[:, j:j + 1]
    out_ref[...] = acc


def _run_combine(vrows, fw):
    return pl.pallas_call(
        _combine_body,
        grid=(T // TB,),
        in_specs=[
            pl.BlockSpec((TB, FK * D), lambda i: (i, 0)),
            pl.BlockSpec((TB, FK), lambda i: (i, 0)),
        ],
        out_specs=pl.BlockSpec((TB, D), lambda i: (i, 0)),
        out_shape=jax.ShapeDtypeStruct((T, D), jnp.float32),
    )(vrows, fw)


def kernel(x, h, router_weight, W_q, K_all, V_all):
    x2 = x.reshape(T, D)
    h2 = h.reshape(T, R)
    wqt = W_q.T
    logits, selflat = _run_router(x2, router_weight)
    pool = _sc_gather(logits.reshape(T * NT, TILE), selflat.reshape(-1),
                      T * SEL_T, TILE)
    cs, ci, q = _run_coarse(pool.reshape(T, POOL), selflat, h2, wqt)
    kc = _sc_gather(K_all.reshape(NK // 2, 2 * R), ci.reshape(-1) // 2,
                    T * CK, 2 * R)
    fw, fi = _run_fine(q, kc.reshape(T, CK * 2 * R), ci)
    vrows = _sc_gather(V_all, fi.reshape(-1), T * FK, D)
    out = _run_combine(vrows.reshape(T, FK * D), fw)
    return (out.reshape(B, S, D), cs.reshape(B, S, CK), ci.reshape(B, S, CK),
            fi.reshape(B, S, FK), fw.reshape(B, S, FK))
